# baseline (speedup 1.0000x reference)
import functools

import jax
import jax.numpy as jnp
from jax import lax
from jax.experimental import pallas as pl
from jax.experimental.pallas import tpu as pltpu
from jax.experimental.pallas import tpu_sc as plsc

LANES = 16
GATHER_SUB = 128


def _make_sc_kernel(B, L, V, D):
    info = plsc.get_sparse_core_info()
    num_workers = info.num_cores * info.num_subcores
    assert B % num_workers == 0
    b_per_w = B // num_workers
    G = 2
    assert b_per_w % G == 0
    nchunks = b_per_w // G
    CH = G * L
    NBUF = 2
    DB = D // LANES
    U = 4
    assert L % U == 0
    subs = []
    off = 0
    while off < CH:
        n = min(GATHER_SUB, CH - off)
        subs.append((off, n))
        off += n
    inv_L = 1.0 / L

    mesh = plsc.VectorSubcoreMesh(core_axis_name="c", subcore_axis_name="s")

    @functools.partial(
        pl.kernel,
        mesh=mesh,
        out_type=jax.ShapeDtypeStruct((B, 2 * D), jnp.float32),
        compiler_params=pltpu.CompilerParams(use_tc_tiling_on_sc=False),
        scratch_types=[
            pltpu.VMEM((b_per_w * L,), jnp.int32),
            pltpu.VMEM((NBUF, CH, D), jnp.float32),
            pltpu.VMEM((b_per_w, 2 * D), jnp.float32),
            pltpu.SemaphoreType.DMA,
            pltpu.SemaphoreType.DMA,
        ],
    )
    def sc_kernel(x_hbm, table_hbm, out_hbm, idx_v, rows_v, out_v, sem0, sem1):
        sems = (sem0, sem1)
        wid = lax.axis_index("s") * info.num_cores + lax.axis_index("c")
        pltpu.sync_copy(x_hbm.at[pl.ds(wid * (b_per_w * L), b_per_w * L)], idx_v)

        def start_gather(c, buf):
            base = c * CH
            for off, n in subs:
                pltpu.make_async_copy(
                    table_hbm.at[idx_v.at[pl.ds(base + off, n)]],
                    rows_v.at[buf, pl.ds(off, n)],
                    sems[buf],
                ).start()

        def wait_gather(c, buf):
            pltpu.make_async_copy(
                table_hbm.at[idx_v.at[pl.ds(c * CH, CH)]],
                rows_v.at[buf],
                sems[buf],
            ).wait()

        def reduce_chunk(c, buf):
            for g in range(G):
                base = g * L
                init = []
                for db in range(DB):
                    v = rows_v[buf, base, pl.ds(db * LANES, LANES)]
                    init.extend((v, v))

                def body(i, carry, base=base, buf=buf):
                    acc = list(carry)
                    r0 = base + 1 + i * U
                    for u in range(U):
                        for db in range(DB):
                            v = rows_v[buf, r0 + u, pl.ds(db * LANES, LANES)]
                            acc[2 * db] = jnp.maximum(acc[2 * db], v)
                            acc[2 * db + 1] = acc[2 * db + 1] + v
                    return tuple(acc)

                n_loop = (L - 1) // U
                acc = lax.fori_loop(0, n_loop, body, tuple(init))
                acc = list(acc)
                for r in range(1 + n_loop * U, L):
                    for db in range(DB):
                        v = rows_v[buf, base + r, pl.ds(db * LANES, LANES)]
                        acc[2 * db] = jnp.maximum(acc[2 * db], v)
                        acc[2 * db + 1] = acc[2 * db + 1] + v
                row = c * G + g
                for db in range(DB):
                    out_v[row, pl.ds(db * LANES, LANES)] = acc[2 * db]
                    out_v[row, pl.ds(D + db * LANES, LANES)] = acc[2 * db + 1] * inv_L

        for b in range(NBUF):
            start_gather(b, b)

        def chunk_body(c0, carry):
            for b in range(NBUF):
                c = c0 * NBUF + b
                wait_gather(c, b)
                reduce_chunk(c, b)

                @pl.when(c + NBUF < nchunks)
                def _start(c=c, b=b):
                    start_gather(c + NBUF, b)

            return carry

        lax.fori_loop(0, nchunks // NBUF, chunk_body, 0)
        pltpu.sync_copy(out_v, out_hbm.at[pl.ds(wid * b_per_w, b_per_w)])

    return sc_kernel


def kernel(x, table):
    B, L = x.shape
    V, D = table.shape
    sc = _make_sc_kernel(B, L, V, D)
    return sc(x.reshape(B * L), table)


# speedup vs baseline: 1.0030x; 1.0030x over previous
import functools

import jax
import jax.numpy as jnp
from jax import lax
from jax.experimental import pallas as pl
from jax.experimental.pallas import tpu as pltpu
from jax.experimental.pallas import tpu_sc as plsc

LANES = 16
GATHER_SUB = 128


def _make_sc_kernel(B, L, V, D):
    info = plsc.get_sparse_core_info()
    num_workers = info.num_cores * info.num_subcores
    assert B % num_workers == 0
    b_per_w = B // num_workers
    G = 2
    assert b_per_w % G == 0
    nchunks = b_per_w // G
    CH = G * L
    NBUF = 2
    DB = D // LANES
    U = 4
    assert L % U == 0
    subs = []
    off = 0
    while off < CH:
        n = min(GATHER_SUB, CH - off)
        subs.append((off, n))
        off += n
    inv_L = 1.0 / L

    mesh = plsc.VectorSubcoreMesh(core_axis_name="c", subcore_axis_name="s")

    @functools.partial(
        pl.kernel,
        mesh=mesh,
        out_type=jax.ShapeDtypeStruct((B, 2 * D), jnp.float32),
        compiler_params=pltpu.CompilerParams(use_tc_tiling_on_sc=False),
        scratch_types=[
            pltpu.VMEM((b_per_w * L,), jnp.int32),
            pltpu.VMEM((NBUF, CH, D), jnp.float32),
            pltpu.VMEM((b_per_w, 2 * D), jnp.float32),
            pltpu.SemaphoreType.DMA,
            pltpu.SemaphoreType.DMA,
        ],
    )
    def sc_kernel(x_hbm, table_hbm, out_hbm, idx_v, rows_v, out_v, sem0, sem1):
        sems = (sem0, sem1)
        wid = lax.axis_index("s") * info.num_cores + lax.axis_index("c")
        pltpu.sync_copy(x_hbm.at[pl.ds(wid * (b_per_w * L), b_per_w * L)], idx_v)

        def start_gather(c, buf):
            base = c * CH
            for off, n in subs:
                pltpu.make_async_copy(
                    table_hbm.at[idx_v.at[pl.ds(base + off, n)]],
                    rows_v.at[buf, pl.ds(off, n)],
                    sems[buf],
                ).start()

        def wait_gather(c, buf):
            pltpu.make_async_copy(
                table_hbm.at[idx_v.at[pl.ds(c * CH, CH)]],
                rows_v.at[buf],
                sems[buf],
            ).wait()

        def reduce_chunk(c, buf):
            for g in range(G):
                base = g * L
                init = []
                for db in range(DB):
                    v = rows_v[buf, base, pl.ds(db * LANES, LANES)]
                    init.extend((v, v))

                def body(i, carry, base=base, buf=buf):
                    acc = list(carry)
                    r0 = base + 1 + i * U
                    for u in range(U):
                        for db in range(DB):
                            v = rows_v[buf, r0 + u, pl.ds(db * LANES, LANES)]
                            acc[2 * db] = jnp.maximum(acc[2 * db], v)
                            acc[2 * db + 1] = acc[2 * db + 1] + v
                    return tuple(acc)

                n_loop = (L - 1) // U
                acc = lax.fori_loop(0, n_loop, body, tuple(init))
                acc = list(acc)
                for r in range(1 + n_loop * U, L):
                    for db in range(DB):
                        v = rows_v[buf, base + r, pl.ds(db * LANES, LANES)]
                        acc[2 * db] = jnp.maximum(acc[2 * db], v)
                        acc[2 * db + 1] = acc[2 * db + 1] + v
                row = c * G + g
                for db in range(DB):
                    out_v[row, pl.ds(db * LANES, LANES)] = acc[2 * db]
                    out_v[row, pl.ds(D + db * LANES, LANES)] = acc[2 * db + 1] * inv_L

        for b in range(NBUF):
            start_gather(b, b)

        def chunk_body(c0, carry):
            for b in range(NBUF):
                c = c0 * NBUF + b
                wait_gather(c, b)
                reduce_chunk(c, b)

                @pl.when(c + NBUF < nchunks)
                def _start(c=c, b=b):
                    start_gather(c + NBUF, b)

            return carry

        lax.fori_loop(0, nchunks // NBUF, chunk_body, 0)
        pltpu.sync_copy(out_v, out_hbm.at[pl.ds(wid * b_per_w, b_per_w)])

    return sc_kernel


def kernel(x, table):
    B, L = x.shape
    V, D = table.shape
    sc = _make_sc_kernel(B, L, V, D)
    t = jax.lax.optimization_barrier(table.reshape(V * D // 128, 128))
    table_lin = t.reshape(V, D)
    return sc(x.reshape(B * L), table_lin)


# speedup vs baseline: 1.5833x; 1.5785x over previous
import functools

import jax
import jax.numpy as jnp
from jax import lax
from jax.experimental import pallas as pl
from jax.experimental.pallas import tpu as pltpu
from jax.experimental.pallas import tpu_sc as plsc

LANES = 16
GATHER_SUB = 128


def _make_sc_kernel(B, L, V, D):
    info = plsc.get_sparse_core_info()
    num_workers = info.num_cores * info.num_subcores
    assert B % num_workers == 0
    b_per_w = B // num_workers
    G = 2
    assert b_per_w % G == 0
    nchunks = b_per_w // G
    CH = G * L
    NBUF = 2
    DB = D // LANES
    U = 4
    assert L % U == 0
    subs = []
    off = 0
    while off < CH:
        n = min(GATHER_SUB, CH - off)
        subs.append((off, n))
        off += n
    inv_L = 1.0 / L

    mesh = plsc.VectorSubcoreMesh(core_axis_name="c", subcore_axis_name="s")

    @functools.partial(
        pl.kernel,
        mesh=mesh,
        out_type=jax.ShapeDtypeStruct((B, 2 * D), jnp.float32),
        compiler_params=pltpu.CompilerParams(use_tc_tiling_on_sc=False),
        scratch_types=[
            pltpu.VMEM((b_per_w * L,), jnp.int32),
            pltpu.VMEM((NBUF, CH, D), jnp.float32),
            pltpu.VMEM((b_per_w, 2 * D), jnp.float32),
            pltpu.SemaphoreType.DMA,
            pltpu.SemaphoreType.DMA,
        ],
    )
    def sc_kernel(x_hbm, table_hbm, out_hbm, idx_v, rows_v, out_v, sem0, sem1):
        sems = (sem0, sem1)
        wid = lax.axis_index("s") * info.num_cores + lax.axis_index("c")
        pltpu.sync_copy(x_hbm.at[pl.ds(wid * (b_per_w * L), b_per_w * L)], idx_v)

        def start_gather(c, buf):
            base = c * CH
            for off, n in subs:
                pltpu.make_async_copy(
                    table_hbm.at[idx_v.at[pl.ds(base + off, n)]],
                    rows_v.at[buf, pl.ds(off, n)],
                    sems[buf],
                ).start()

        def wait_gather(c, buf):
            pltpu.make_async_copy(
                table_hbm.at[idx_v.at[pl.ds(c * CH, CH)]],
                rows_v.at[buf],
                sems[buf],
            ).wait()

        def reduce_chunk(c, buf):
            for g in range(G):
                base = g * L
                init = []
                for db in range(DB):
                    v = rows_v[buf, base, pl.ds(db * LANES, LANES)]
                    init.extend((v, v))

                def body(i, carry, base=base, buf=buf):
                    acc = list(carry)
                    r0 = base + 1 + i * U
                    for u in range(U):
                        for db in range(DB):
                            v = rows_v[buf, r0 + u, pl.ds(db * LANES, LANES)]
                            acc[2 * db] = jnp.maximum(acc[2 * db], v)
                            acc[2 * db + 1] = acc[2 * db + 1] + v
                    return tuple(acc)

                n_loop = (L - 1) // U
                acc = lax.fori_loop(0, n_loop, body, tuple(init))
                acc = list(acc)
                for r in range(1 + n_loop * U, L):
                    for db in range(DB):
                        v = rows_v[buf, base + r, pl.ds(db * LANES, LANES)]
                        acc[2 * db] = jnp.maximum(acc[2 * db], v)
                        acc[2 * db + 1] = acc[2 * db + 1] + v
                row = c * G + g
                for db in range(DB):
                    out_v[row, pl.ds(db * LANES, LANES)] = acc[2 * db]
                    out_v[row, pl.ds(D + db * LANES, LANES)] = acc[2 * db + 1] * inv_L

        for b in range(NBUF):
            start_gather(b, b)

        def chunk_body(c0, carry):
            for b in range(NBUF):
                c = c0 * NBUF + b
                wait_gather(c, b)
                reduce_chunk(c, b)

                @pl.when(c + NBUF < nchunks)
                def _start(c=c, b=b):
                    start_gather(c + NBUF, b)

            return carry

        lax.fori_loop(0, nchunks // NBUF, chunk_body, 0)
        pltpu.sync_copy(out_v, out_hbm.at[pl.ds(wid * b_per_w, b_per_w)])

    return sc_kernel


def kernel(x, table):
    B, L = x.shape
    V, D = table.shape
    CB = 4096
    HALF = CB // 2
    grid = -(-V // CB)
    Vpad = grid * CB
    tableT = table.T

    def tc_body(t_ref, o_ref):
        t = t_ref[...].T
        o_ref[...] = jnp.concatenate([t[:HALF], t[HALF:]], axis=1)

    relaid = pl.pallas_call(
        tc_body,
        grid=(grid,),
        in_specs=[pl.BlockSpec((D, CB), lambda i: (0, i))],
        out_specs=pl.BlockSpec((HALF, 2 * D), lambda i: (i, 0)),
        out_shape=jax.ShapeDtypeStruct((grid * HALF, 2 * D), jnp.float32),
    )(tableT)
    table_lin = relaid.reshape(Vpad, D)
    g = (x & ~(CB - 1)) | ((x & (HALF - 1)) << 1) | ((x >> 11) & 1)
    sc = _make_sc_kernel(B, L, Vpad, D)
    return sc(g.reshape(B * L), table_lin)
